# Initial kernel scaffold; baseline (speedup 1.0000x reference)
import jax
import jax.numpy as jnp
from jax.experimental import pallas as pl


def kernel(x, w_router, w_expert, b_expert):
    raise NotImplementedError("write your pallas kernel here")


# speedup vs baseline: 4.9238x; 4.9238x over previous
import jax
import jax.numpy as jnp
from jax.experimental import pallas as pl
from jax.experimental.pallas import tpu as pltpu

_BLOCK = 1024


def _body(x_ref, wr_ref, we_ref, be_ref, o_ref):
    x = x_ref[...]
    e = we_ref.shape[0]
    logits = jnp.dot(x, wr_ref[...], preferred_element_type=jnp.float32)
    values = jnp.argmax(logits, axis=-1)
    iota = jax.lax.broadcasted_iota(jnp.int32, (1, e), 1)
    onehot = (values[:, None] == iota).astype(jnp.float32)
    w_tok = jnp.dot(onehot, we_ref[...], preferred_element_type=jnp.float32)
    b_tok = jnp.dot(onehot, be_ref[...], preferred_element_type=jnp.float32)
    o_ref[...] = jnp.maximum(x * w_tok + b_tok, 0.0)


def kernel(x, w_router, w_expert, b_expert):
    n, d = x.shape
    e = w_router.shape[1]
    block = min(_BLOCK, n)
    return pl.pallas_call(
        _body,
        grid=(n // block,),
        in_specs=[
            pl.BlockSpec((block, d), lambda i: (i, 0)),
            pl.BlockSpec((d, e), lambda i: (0, 0)),
            pl.BlockSpec((e, d), lambda i: (0, 0)),
            pl.BlockSpec((e, d), lambda i: (0, 0)),
        ],
        out_specs=pl.BlockSpec((block, d), lambda i: (i, 0)),
        out_shape=jax.ShapeDtypeStruct((n, d), jnp.float32),
        compiler_params=pltpu.CompilerParams(
            dimension_semantics=("arbitrary",),
        ),
    )(x, w_router, w_expert, b_expert)


# speedup vs baseline: 5.5670x; 1.1306x over previous
import jax
import jax.numpy as jnp
from jax.experimental import pallas as pl
from jax.experimental.pallas import tpu as pltpu

_BLOCK = 2048


def _body(x_ref, wr_ref, we_ref, be_ref, o_ref):
    x = x_ref[...]
    e = we_ref.shape[0]
    logits = jnp.dot(x, wr_ref[...], preferred_element_type=jnp.float32)
    values = jnp.argmax(logits, axis=-1)
    iota = jax.lax.broadcasted_iota(jnp.int32, (1, e), 1)
    onehot = (values[:, None] == iota).astype(jnp.float32)
    w_tok = jnp.dot(onehot, we_ref[...], preferred_element_type=jnp.float32)
    b_tok = jnp.dot(onehot, be_ref[...], preferred_element_type=jnp.float32)
    o_ref[...] = jnp.maximum(x * w_tok + b_tok, 0.0)


def kernel(x, w_router, w_expert, b_expert):
    n, d = x.shape
    e = w_router.shape[1]
    block = min(_BLOCK, n)
    return pl.pallas_call(
        _body,
        grid=(n // block,),
        in_specs=[
            pl.BlockSpec((block, d), lambda i: (i, 0)),
            pl.BlockSpec((d, e), lambda i: (0, 0)),
            pl.BlockSpec((e, d), lambda i: (0, 0)),
            pl.BlockSpec((e, d), lambda i: (0, 0)),
        ],
        out_specs=pl.BlockSpec((block, d), lambda i: (i, 0)),
        out_shape=jax.ShapeDtypeStruct((n, d), jnp.float32),
        compiler_params=pltpu.CompilerParams(
            dimension_semantics=("arbitrary",),
        ),
    )(x, w_router, w_expert, b_expert)


# speedup vs baseline: 5.7910x; 1.0402x over previous
import jax
import jax.numpy as jnp
from jax.experimental import pallas as pl
from jax.experimental.pallas import tpu as pltpu

_BLOCK = 4096


def _body(x_ref, wr_ref, we_ref, be_ref, o_ref):
    x = x_ref[...]
    e = we_ref.shape[0]
    logits = jnp.dot(x, wr_ref[...], preferred_element_type=jnp.float32)
    values = jnp.argmax(logits, axis=-1)
    iota = jax.lax.broadcasted_iota(jnp.int32, (1, e), 1)
    onehot = (values[:, None] == iota).astype(jnp.float32)
    w_tok = jnp.dot(onehot, we_ref[...], preferred_element_type=jnp.float32)
    b_tok = jnp.dot(onehot, be_ref[...], preferred_element_type=jnp.float32)
    o_ref[...] = jnp.maximum(x * w_tok + b_tok, 0.0)


def kernel(x, w_router, w_expert, b_expert):
    n, d = x.shape
    e = w_router.shape[1]
    block = min(_BLOCK, n)
    return pl.pallas_call(
        _body,
        grid=(n // block,),
        in_specs=[
            pl.BlockSpec((block, d), lambda i: (i, 0)),
            pl.BlockSpec((d, e), lambda i: (0, 0)),
            pl.BlockSpec((e, d), lambda i: (0, 0)),
            pl.BlockSpec((e, d), lambda i: (0, 0)),
        ],
        out_specs=pl.BlockSpec((block, d), lambda i: (i, 0)),
        out_shape=jax.ShapeDtypeStruct((n, d), jnp.float32),
        compiler_params=pltpu.CompilerParams(
            dimension_semantics=("arbitrary",),
        ),
    )(x, w_router, w_expert, b_expert)


# speedup vs baseline: 5.7938x; 1.0005x over previous
import jax
import jax.numpy as jnp
from jax.experimental import pallas as pl
from jax.experimental.pallas import tpu as pltpu

_BLOCK = 4096


def _body(x_ref, wr_ref, we_ref, be_ref, o_ref):
    x = x_ref[...]
    e = we_ref.shape[0]
    logits = jnp.dot(x, wr_ref[...], preferred_element_type=jnp.float32)
    values = jnp.argmax(logits, axis=-1)
    iota = jax.lax.broadcasted_iota(jnp.int32, (1, e), 1)
    onehot = (values[:, None] == iota).astype(jnp.float32)
    w_tok = jnp.dot(onehot, we_ref[...], preferred_element_type=jnp.float32)
    b_tok = jnp.dot(onehot, be_ref[...], preferred_element_type=jnp.float32)
    o_ref[...] = jnp.maximum(x * w_tok + b_tok, 0.0)


def kernel(x, w_router, w_expert, b_expert):
    n, d = x.shape
    e = w_router.shape[1]
    block = min(_BLOCK, n)
    return pl.pallas_call(
        _body,
        grid=(n // block,),
        in_specs=[
            pl.BlockSpec((block, d), lambda i: (i, 0)),
            pl.BlockSpec((d, e), lambda i: (0, 0)),
            pl.BlockSpec((e, d), lambda i: (0, 0)),
            pl.BlockSpec((e, d), lambda i: (0, 0)),
        ],
        out_specs=pl.BlockSpec((block, d), lambda i: (i, 0)),
        out_shape=jax.ShapeDtypeStruct((n, d), jnp.float32),
        compiler_params=pltpu.CompilerParams(
            dimension_semantics=("parallel",),
        ),
    )(x, w_router, w_expert, b_expert)
